# Initial kernel scaffold; baseline (speedup 1.0000x reference)
import jax
import jax.numpy as jnp
from jax.experimental import pallas as pl


def kernel(x, outmask, t):
    raise NotImplementedError("write your pallas kernel here")


# speedup vs baseline: 13.4485x; 13.4485x over previous
import functools

import jax
import jax.numpy as jnp
from jax import lax
from jax.experimental import pallas as pl
from jax.experimental.pallas import tpu as pltpu
from jax.experimental.pallas import tpu_sc as plsc

B = 1024
V = 100000
NCAND = 50
NCAND_PAD = 64
L = 16

_info = plsc.get_sparse_core_info()
_NC, _NS = _info.num_cores, _info.num_subcores
NW = _NC * _NS
ROWS_PER_W = B // NW


def _sc_body(idx_hbm, xt_hbm, x0_hbm, x1_hbm, out_hbm, idx_v, row_v, xbuf):
    c = lax.axis_index("c")
    s = lax.axis_index("s")
    wid = s * _NC + c
    base = wid * ROWS_PER_W

    pltpu.sync_copy(xt_hbm.at[0, pl.ds(base, ROWS_PER_W)], xbuf)
    pltpu.sync_copy(xbuf, x0_hbm.at[pl.ds(base, ROWS_PER_W)])
    pltpu.sync_copy(xt_hbm.at[1, pl.ds(base, ROWS_PER_W)], xbuf)
    pltpu.sync_copy(xbuf, x1_hbm.at[pl.ds(base, ROWS_PER_W)])

    pltpu.sync_copy(
        idx_hbm.at[pl.ds(base * NCAND_PAD, ROWS_PER_W * NCAND_PAD)], idx_v
    )

    zeros16 = jnp.zeros((L,), jnp.float32)
    ones16 = jnp.ones((L,), jnp.float32)

    def zbody(i, carry):
        row_v[pl.ds(i * L, L)] = zeros16
        return carry

    lax.fori_loop(0, V // L, zbody, 0, unroll=8)

    def rbody(r, carry):
        rbase = r * NCAND_PAD
        for j in range(NCAND_PAD // L):
            iv = idx_v[pl.ds(rbase + j * L, L)]
            plsc.store_scatter(row_v, [iv - 1], ones16, mask=iv > 0)
        pltpu.sync_copy(row_v, out_hbm.at[base + r])
        for j in range(NCAND_PAD // L):
            iv = idx_v[pl.ds(rbase + j * L, L)]
            plsc.store_scatter(row_v, [iv - 1], zeros16, mask=iv > 0)
        return carry

    lax.fori_loop(0, ROWS_PER_W, rbody, 0)


@jax.jit
def _teacher_force(x, outmask, t):
    idx = lax.dynamic_index_in_dim(outmask, t, 1, keepdims=False)
    idx = idx[:, 2:]
    idx = jnp.concatenate(
        [idx, jnp.zeros((B, NCAND_PAD - NCAND), jnp.int32)], axis=1
    )
    idx_flat = idx.reshape(-1)
    x_t = lax.dynamic_index_in_dim(x, t, 2, keepdims=False)

    mesh = plsc.VectorSubcoreMesh(core_axis_name="c", subcore_axis_name="s")
    k = pl.kernel(
        _sc_body,
        mesh=mesh,
        out_type=(
            jax.ShapeDtypeStruct((B, 128), jnp.float32),
            jax.ShapeDtypeStruct((B, 128), jnp.float32),
            jax.ShapeDtypeStruct((B, V), jnp.float32),
        ),
        scratch_types=[
            pltpu.VMEM((ROWS_PER_W * NCAND_PAD,), jnp.int32),
            pltpu.VMEM((V,), jnp.float32),
            pltpu.VMEM((ROWS_PER_W, 128), jnp.float32),
        ],
        compiler_params=pltpu.CompilerParams(needs_layout_passes=False),
    )
    x0, x1, outmask_t = k(idx_flat, x_t)
    return x0, x1, outmask_t


def kernel(x, outmask, t):
    return _teacher_force(x, outmask, t)


# speedup vs baseline: 40.6719x; 3.0243x over previous
import jax
import jax.numpy as jnp
from jax import lax
from jax.experimental import pallas as pl
from jax.experimental.pallas import tpu as pltpu
from jax.experimental.pallas import tpu_sc as plsc

B = 1024
V = 100000
NCAND = 50
NCAND_PAD = 64
L = 16

_info = plsc.get_sparse_core_info()
_NC, _NS = _info.num_cores, _info.num_subcores
NW = _NC * _NS

N_SLABS = 4
N_STRIPES = 8
SLAB_V = V // N_SLABS
STRIPE_B = B // N_STRIPES
CV = 200
NCH = SLAB_V // CV
CHW = CV * STRIPE_B
STRIPE_IDX = STRIPE_B * NCAND_PAD
LST_CAP = STRIPE_IDX + L
SENTINEL = 2**30
XROWS = B // NW


def _zero_buf(buf):
    zeros16 = jnp.zeros((L,), jnp.float32)

    def zb(k, carry):
        buf[k >> 3, pl.ds((k & 7) * L, L)] = zeros16
        return carry

    lax.fori_loop(0, CV * (STRIPE_B // L), zb, 0, unroll=8)


def _sc_body(idx_hbm, xt_hbm, x0_hbm, x1_hbm, outT_hbm,
             idx_stage, lst, hits0, hits1, buf0, buf1, xbuf, sem0, sem1):
    c = lax.axis_index("c")
    s = lax.axis_index("s")
    wid = s * _NC + c
    stripe = wid % N_STRIPES
    slab = wid // N_STRIPES
    vlo = slab * SLAB_V

    xbase = wid * XROWS
    pltpu.sync_copy(xt_hbm.at[0, pl.ds(xbase, XROWS)], xbuf)
    pltpu.sync_copy(xbuf, x0_hbm.at[pl.ds(xbase, XROWS)])
    pltpu.sync_copy(xt_hbm.at[1, pl.ds(xbase, XROWS)], xbuf)
    pltpu.sync_copy(xbuf, x1_hbm.at[pl.ds(xbase, XROWS)])

    pltpu.sync_copy(idx_hbm.at[pl.ds(stripe * STRIPE_IDX, STRIPE_IDX)],
                    idx_stage)

    iota16 = lax.iota(jnp.int32, L)
    ones16 = jnp.ones((L,), jnp.float32)
    zeros16 = jnp.zeros((L,), jnp.float32)
    sent16 = jnp.full((L,), SENTINEL, jnp.int32)

    _zero_buf(buf0)
    _zero_buf(buf1)

    def fbody(j, cnt):
        iv = idx_stage[pl.ds(j * L, L)]
        v = iv - 1
        m = (iv > 0) & (v >= vlo) & (v < vlo + SLAB_V)
        b_local = (j * L + iota16) >> 6
        off = (v - vlo) * STRIPE_B + b_local
        plsc.store_compressed(lst.at[pl.ds(cnt, L)], off, mask=m)
        return cnt + jnp.sum(m.astype(jnp.int32))

    cnt = lax.fori_loop(0, STRIPE_IDX // L, fbody, jnp.int32(0))
    lst[pl.ds(cnt, L)] = sent16
    n_iter = (cnt + (L - 1)) >> 4

    dst_b = stripe * STRIPE_B

    def chunk_dst(ci):
        return outT_hbm.at[pl.ds(vlo + ci * CV, CV), pl.ds(dst_b, STRIPE_B)]

    def do_chunk(ci, buf, hits, sem, h_in):
        @pl.when(ci >= 2)
        def _():
            pltpu.make_async_copy(buf, chunk_dst(ci - 2), sem).wait()

        def ub(j, carry):
            e = hits[pl.ds(j * L, L)]
            plsc.store_scatter(buf, [e >> 7, e & (STRIPE_B - 1)], zeros16,
                               mask=e < CHW)
            return carry

        lax.fori_loop(0, (h_in + (L - 1)) >> 4, ub, 0)

        lo = ci * CHW

        def sb(j, hcnt):
            e = lst[pl.ds(j * L, L)]
            rel = e - lo
            m = (rel >= 0) & (rel < CHW)
            plsc.store_scatter(buf, [rel >> 7, rel & (STRIPE_B - 1)], ones16,
                               mask=m)
            plsc.store_compressed(hits.at[pl.ds(hcnt, L)], rel, mask=m)
            return hcnt + jnp.sum(m.astype(jnp.int32))

        h_out = lax.fori_loop(0, n_iter, sb, jnp.int32(0))
        hits[pl.ds(h_out, L)] = sent16
        pltpu.make_async_copy(buf, chunk_dst(ci), sem).start()
        return h_out

    def pair(i, carry):
        h0, h1 = carry
        h0 = do_chunk(2 * i, buf0, hits0, sem0, h0)
        h1 = do_chunk(2 * i + 1, buf1, hits1, sem1, h1)
        return h0, h1

    h0, h1 = lax.fori_loop(0, NCH // 2, pair, (jnp.int32(0), jnp.int32(0)))
    do_chunk(jnp.int32(NCH - 1), buf0, hits0, sem0, h0)
    pltpu.make_async_copy(buf0, chunk_dst(jnp.int32(NCH - 1)), sem0).wait()
    pltpu.make_async_copy(buf1, chunk_dst(jnp.int32(NCH - 2)), sem1).wait()


@jax.jit
def _teacher_force(x, outmask, t):
    idx = lax.dynamic_index_in_dim(outmask, t, 1, keepdims=False)
    idx = idx[:, 2:]
    idx = jnp.concatenate(
        [idx, jnp.zeros((B, NCAND_PAD - NCAND), jnp.int32)], axis=1
    )
    idx_flat = idx.reshape(-1)
    x_t = lax.dynamic_index_in_dim(x, t, 2, keepdims=False)

    mesh = plsc.VectorSubcoreMesh(core_axis_name="c", subcore_axis_name="s")
    k = pl.kernel(
        _sc_body,
        mesh=mesh,
        out_type=(
            jax.ShapeDtypeStruct((B, 128), jnp.float32),
            jax.ShapeDtypeStruct((B, 128), jnp.float32),
            jax.ShapeDtypeStruct((V, B), jnp.float32),
        ),
        scratch_types=[
            pltpu.VMEM((STRIPE_IDX,), jnp.int32),
            pltpu.VMEM((LST_CAP,), jnp.int32),
            pltpu.VMEM((LST_CAP,), jnp.int32),
            pltpu.VMEM((LST_CAP,), jnp.int32),
            pltpu.VMEM((CV, STRIPE_B), jnp.float32),
            pltpu.VMEM((CV, STRIPE_B), jnp.float32),
            pltpu.VMEM((XROWS, 128), jnp.float32),
            pltpu.SemaphoreType.DMA,
            pltpu.SemaphoreType.DMA,
        ],
        compiler_params=pltpu.CompilerParams(needs_layout_passes=False),
    )
    x0, x1, outT = k(idx_flat, x_t)
    return x0, x1, outT.T


def kernel(x, outmask, t):
    return _teacher_force(x, outmask, t)
